# baseline (speedup 1.0000x reference)
import functools

import jax
import jax.numpy as jnp
from jax import lax
from jax.experimental import pallas as pl
from jax.experimental.pallas import tpu as pltpu
from jax.experimental.pallas import tpu_sc as plsc

N_CLASSES = 1000000
EMBED_SIZE = 64
BATCH = 16384

_NC = 2
_NS = 16
_NW = _NC * _NS
_B_PER_W = BATCH // _NW
_NSLOTS = 14


def _make_kernel():
    mesh = plsc.VectorSubcoreMesh(core_axis_name="c", subcore_axis_name="s")

    @functools.partial(
        pl.kernel,
        mesh=mesh,
        out_type=jax.ShapeDtypeStruct((EMBED_SIZE, BATCH), jnp.float32),
        compiler_params=pltpu.CompilerParams(needs_layout_passes=False),
        scratch_types=[
            pltpu.VMEM((_B_PER_W + 16,), jnp.int32),
            pltpu.VMEM((_NSLOTS, EMBED_SIZE, 128), jnp.float32),
            pltpu.VMEM((EMBED_SIZE, 128), jnp.float32),
        ]
        + [pltpu.SemaphoreType.DMA] * _NSLOTS,
    )
    def embed(labels_hbm, table_t_hbm, out_t_hbm, idx_v, blk_v, colq_v, *sems):
        wid = lax.axis_index("s") * _NC + lax.axis_index("c")
        base = wid * _B_PER_W
        pltpu.sync_copy(labels_hbm.at[pl.ds(base, _B_PER_W)], idx_v.at[pl.ds(0, _B_PER_W)])

        iota16 = lax.iota(jnp.int32, 16)

        def fire(p, slot):
            v = idx_v[pl.ds(p, 16)]
            grp = v[0] >> 7
            start = pl.multiple_of(grp * 128, 128)
            for tr in range(0, EMBED_SIZE, 8):
                pltpu.async_copy(
                    table_t_hbm.at[pl.ds(tr, 8), pl.ds(start, 128)],
                    blk_v.at[slot, pl.ds(tr, 8), :],
                    sems[slot],
                )

        def drain(slot):
            for tr in range(0, EMBED_SIZE, 8):
                pltpu.make_async_copy(
                    table_t_hbm.at[pl.ds(tr, 8), pl.ds(0, 128)],
                    blk_v.at[slot, pl.ds(tr, 8), :],
                    sems[slot],
                ).wait()

        def extract(p, slot):
            v = idx_v[pl.ds(p, 16)]
            colv = jnp.full((16,), v[0] & 127, jnp.int32)
            ocols = jnp.full((16,), p & 127, jnp.int32)
            for e0 in range(0, EMBED_SIZE, 16):
                rows = jnp.full((16,), e0, jnp.int32) + iota16
                vals = plsc.load_gather(blk_v.at[slot], [rows, colv])
                plsc.store_scatter(colq_v, [rows, ocols], vals)

        for b in range(_NSLOTS):
            fire(b, b)

        def body(t, _):
            for b in range(_NSLOTS):
                p = t * _NSLOTS + b

                @pl.when(p < _B_PER_W)
                def _():
                    drain(b)
                    extract(p, b)

                    @pl.when(p + _NSLOTS < _B_PER_W)
                    def _():
                        fire(p + _NSLOTS, b)

                    @pl.when((p & 127) == 127)
                    def _():
                        q = p >> 7
                        off = pl.multiple_of(base + q * 128, 128)
                        pltpu.sync_copy(colq_v, out_t_hbm.at[:, pl.ds(off, 128)])

            return ()

        lax.fori_loop(0, (_B_PER_W + _NSLOTS - 1) // _NSLOTS, body, ())

    return embed


_embed = jax.jit(_make_kernel())


def kernel(labels, table):
    out_t = _embed(labels, table.T)
    return out_t.T


# speedup vs baseline: 1.0052x; 1.0052x over previous
import functools

import jax
import jax.numpy as jnp
from jax import lax
from jax.experimental import pallas as pl
from jax.experimental.pallas import tpu as pltpu
from jax.experimental.pallas import tpu_sc as plsc

N_CLASSES = 1000000
EMBED_SIZE = 64
BATCH = 16384

_NC = 2
_NS = 16
_NW = _NC * _NS
_B_PER_W = BATCH // _NW
_NSLOTS = 14


def _make_kernel():
    mesh = plsc.VectorSubcoreMesh(core_axis_name="c", subcore_axis_name="s")

    @functools.partial(
        pl.kernel,
        mesh=mesh,
        out_type=jax.ShapeDtypeStruct((EMBED_SIZE, BATCH), jnp.float32),
        compiler_params=pltpu.CompilerParams(needs_layout_passes=False),
        scratch_types=[
            pltpu.VMEM((_B_PER_W + 16,), jnp.int32),
            pltpu.VMEM((_NSLOTS, EMBED_SIZE, 128), jnp.float32),
            pltpu.VMEM((EMBED_SIZE, 128), jnp.float32),
        ]
        + [pltpu.SemaphoreType.DMA] * _NSLOTS,
    )
    def embed(labels_hbm, table_t_hbm, out_t_hbm, idx_v, blk_v, colq_v, *sems):
        wid = lax.axis_index("s") * _NC + lax.axis_index("c")
        base = wid * _B_PER_W
        pltpu.sync_copy(labels_hbm.at[pl.ds(base, _B_PER_W)], idx_v.at[pl.ds(0, _B_PER_W)])

        iota16 = lax.iota(jnp.int32, 16)

        def fire(p, slot):
            v = idx_v[pl.ds(p, 16)]
            grp = v[0] >> 7
            start = pl.multiple_of(grp * 128, 128)
            pltpu.async_copy(
                table_t_hbm.at[:, pl.ds(start, 128)],
                blk_v.at[slot],
                sems[slot],
            )

        def drain(slot):
            pltpu.make_async_copy(
                table_t_hbm.at[:, pl.ds(0, 128)],
                blk_v.at[slot],
                sems[slot],
            ).wait()

        def extract(p, slot):
            v = idx_v[pl.ds(p, 16)]
            colv = jnp.full((16,), v[0] & 127, jnp.int32)
            ocols = jnp.full((16,), p & 127, jnp.int32)
            for e0 in range(0, EMBED_SIZE, 16):
                rows = jnp.full((16,), e0, jnp.int32) + iota16
                vals = plsc.load_gather(blk_v.at[slot], [rows, colv])
                plsc.store_scatter(colq_v, [rows, ocols], vals)

        for b in range(_NSLOTS):
            fire(b, b)

        def body(t, _):
            for b in range(_NSLOTS):
                p = t * _NSLOTS + b

                @pl.when(p < _B_PER_W)
                def _():
                    drain(b)
                    extract(p, b)

                    @pl.when(p + _NSLOTS < _B_PER_W)
                    def _():
                        fire(p + _NSLOTS, b)

                    @pl.when((p & 127) == 127)
                    def _():
                        q = p >> 7
                        off = pl.multiple_of(base + q * 128, 128)
                        pltpu.sync_copy(colq_v, out_t_hbm.at[:, pl.ds(off, 128)])

            return ()

        lax.fori_loop(0, (_B_PER_W + _NSLOTS - 1) // _NSLOTS, body, ())

    return embed


_embed = jax.jit(_make_kernel())


def kernel(labels, table):
    out_t = _embed(labels, table.T)
    return out_t.T
